# baseline (speedup 1.0000x reference)
import functools

import jax
import jax.numpy as jnp
from jax import lax
from jax.experimental import pallas as pl
from jax.experimental.pallas import tpu as pltpu
from jax.experimental.pallas import tpu_sc as plsc

import numpy as np

TOTAL = 32768
NSEG = 16
NTILES = 16
CHUNK = TOTAL // NTILES
LANES = 16
VREGS = CHUNK // LANES

MINF = np.float32(float("-inf"))
BIG = np.int32(2147483647)


def _body(values_hbm, starts_hbm, ends_hbm, out_hbm,
          vals_v, starts_v, ends_v, my_vals_v, my_idxs_v,
          shared_vals, shared_idxs, merge_vals_v, merge_idxs_v, out_v,
          sem):
    tid = lax.axis_index("s")
    base = tid * CHUNK

    c1 = pltpu.async_copy(values_hbm.at[pl.ds(base, CHUNK)], vals_v, sem)
    c2 = pltpu.async_copy(starts_hbm, starts_v, sem)
    c3 = pltpu.async_copy(ends_hbm, ends_v, sem)
    c1.wait()
    c2.wait()
    c3.wait()

    lane = lax.iota(jnp.int32, LANES)
    starts_vec = starts_v[...]
    ends_vec = ends_v[...]
    my_vals = jnp.full((LANES,), MINF, jnp.float32)
    my_idxs = jnp.full((LANES,), BIG, jnp.int32)
    laneoffs = [lane + np.int32(k * LANES) for k in range(8)]

    for s in range(NSEG):
        lo = starts_vec[s]
        hi = ends_vec[s]
        n0 = jnp.clip(lo - base, 0, CHUNK)
        n1 = jnp.clip(hi - base, 0, CHUNK)

        def _compute(n0=n0, n1=n1, lo=lo, hi=hi):
            first = n0 >> 4
            last = (n1 - 1) >> 4

            def masked(i, bv, bi):
                off = i * LANES
                pos = base + off + lane
                v = vals_v[pl.ds(off, LANES)]
                vm = jnp.where((pos >= lo) & (pos < hi), v, MINF)
                upd = vm > bv
                return jnp.where(upd, vm, bv), jnp.where(upd, pos, bi)

            bv = jnp.full((LANES,), MINF, jnp.float32)
            bi = jnp.full((LANES,), BIG, jnp.int32)
            bv, bi = masked(first, bv, bi)

            def comb(a, b):
                av, ai = a
                bv_, bi_ = b
                upd = bv_ > av
                return jnp.maximum(av, bv_), jnp.where(upd, bi_, ai)

            g0 = first + 1
            n_int = jnp.maximum(last - g0, 0)
            ng = n_int >> 3

            def group(g, carry):
                goff = (g0 + g * 8) * LANES
                gp = base + goff
                leaves = [(vals_v[pl.ds(goff + k * LANES, LANES)],
                           gp + laneoffs[k]) for k in range(8)]
                l2 = [comb(leaves[k], leaves[k + 1]) for k in (0, 2, 4, 6)]
                l3 = [comb(l2[0], l2[1]), comb(l2[2], l2[3])]
                return comb(carry, comb(l3[0], l3[1]))

            bv, bi = lax.fori_loop(0, ng, group, (bv, bi))

            def interior(i, carry):
                cv, ci = carry
                off = i * LANES
                v = vals_v[pl.ds(off, LANES)]
                upd = v > cv
                return (jnp.where(upd, v, cv),
                        jnp.where(upd, base + off + lane, ci))

            bv, bi = lax.fori_loop(g0 + ng * 8, last, interior, (bv, bi))

            def do_last(args):
                bv, bi = args
                return masked(last, bv, bi)

            bv, bi = lax.cond(last > first, do_last, lambda a: a, (bv, bi))

            mx = jnp.max(bv)
            mi = jnp.min(jnp.where(bv == mx, bi, BIG))
            return mx, mi

        mx, mi = lax.cond(
            n1 > n0, _compute,
            lambda: (jnp.float32(MINF), jnp.int32(BIG)))
        my_vals = jnp.where(lane == s, mx, my_vals)
        my_idxs = jnp.where(lane == s, mi, my_idxs)

    my_vals_v[...] = my_vals
    my_idxs_v[...] = my_idxs
    pltpu.sync_copy(my_vals_v, shared_vals.at[pl.ds(tid * LANES, LANES)])
    pltpu.sync_copy(my_idxs_v, shared_idxs.at[pl.ds(tid * LANES, LANES)])
    plsc.subcore_barrier()

    @pl.when(tid == 0)
    def _merge():
        m1 = pltpu.async_copy(shared_vals, merge_vals_v, sem)
        m2 = pltpu.async_copy(shared_idxs, merge_idxs_v, sem)
        m1.wait()
        m2.wait()
        acc_v = merge_vals_v[pl.ds(0, LANES)]
        acc_i = merge_idxs_v[pl.ds(0, LANES)]
        for r in range(1, NTILES):
            row_v = merge_vals_v[pl.ds(r * LANES, LANES)]
            row_i = merge_idxs_v[pl.ds(r * LANES, LANES)]
            upd = row_v > acc_v
            acc_v = jnp.where(upd, row_v, acc_v)
            acc_i = jnp.where(upd, row_i, acc_i)
        out_v[...] = jnp.where(ends_vec > starts_vec, acc_i, BIG)
        pltpu.sync_copy(out_v, out_hbm)


@functools.lru_cache(maxsize=1)
def _build():
  return pl.kernel(
    _body,
    out_type=jax.ShapeDtypeStruct((NSEG,), jnp.int32),
    mesh=plsc.VectorSubcoreMesh(
        core_axis_name="c", subcore_axis_name="s",
        num_cores=1, num_subcores=NTILES),
    scratch_types=[
        pltpu.VMEM((CHUNK,), jnp.float32),
        pltpu.VMEM((NSEG,), jnp.int32),
        pltpu.VMEM((NSEG,), jnp.int32),
        pltpu.VMEM((LANES,), jnp.float32),
        pltpu.VMEM((LANES,), jnp.int32),
        pltpu.VMEM_SHARED((NTILES * LANES,), jnp.float32),
        pltpu.VMEM_SHARED((NTILES * LANES,), jnp.int32),
        pltpu.VMEM((NTILES * LANES,), jnp.float32),
        pltpu.VMEM((NTILES * LANES,), jnp.int32),
        pltpu.VMEM((NSEG,), jnp.int32),
        pltpu.SemaphoreType.DMA,
    ],
    compiler_params=pltpu.CompilerParams(needs_layout_passes=False),
  )


def kernel(values, prefix_sum):
    ps = prefix_sum.astype(jnp.int32)
    starts = jnp.concatenate([jnp.zeros((1,), jnp.int32), ps[:-1]])
    out = _build()(values, starts, ps)
    return out.astype(jnp.int64)


# speedup vs baseline: 1.2673x; 1.2673x over previous
import functools

import jax
import jax.numpy as jnp
from jax import lax
from jax.experimental import pallas as pl
from jax.experimental.pallas import tpu as pltpu
from jax.experimental.pallas import tpu_sc as plsc

import numpy as np

TOTAL = 32768
NSEG = 16
NTILES = 16
CHUNK = TOTAL // NTILES
LANES = 16
VREGS = CHUNK // LANES

MINF = np.float32(float("-inf"))
BIG = np.int32(2147483647)


def _body(values_hbm, starts_hbm, ends_hbm, out_hbm,
          vals_v, starts_v, ends_v, my_vals_v, my_idxs_v,
          shared_vals, shared_idxs, merge_vals_v, merge_idxs_v, out_v,
          sem):
    tid = lax.axis_index("s")
    base = tid * CHUNK

    c1 = pltpu.async_copy(values_hbm.at[pl.ds(base, CHUNK)], vals_v, sem)
    c2 = pltpu.async_copy(starts_hbm, starts_v, sem)
    c3 = pltpu.async_copy(ends_hbm, ends_v, sem)
    c1.wait()
    c2.wait()
    c3.wait()

    lane = lax.iota(jnp.int32, LANES)
    starts_vec = starts_v[...]
    ends_vec = ends_v[...]
    laneoffs = [lane + np.int32(k * LANES) for k in range(8)]

    seg_a = jnp.sum((ends_vec <= base).astype(jnp.int32))
    seg_b = jnp.sum((starts_vec < base + CHUNK).astype(jnp.int32))

    def per_segment(s, carry):
        my_vals, my_idxs = carry
        svec = jnp.full((LANES,), s, jnp.int32)
        lo = plsc.load_gather(starts_v, [svec])[0]
        hi = plsc.load_gather(ends_v, [svec])[0]
        n0 = jnp.clip(lo - base, 0, CHUNK)
        n1 = jnp.clip(hi - base, 0, CHUNK)
        first = jnp.minimum(n0 >> 4, VREGS - 1)
        last = jnp.maximum((n1 - 1) >> 4, 0)

        def masked(i, bv, bi):
            off = i * LANES
            pos = base + off + lane
            v = vals_v[pl.ds(off, LANES)]
            vm = jnp.where((pos >= lo) & (pos < hi), v, MINF)
            upd = vm > bv
            return jnp.where(upd, vm, bv), jnp.where(upd, pos, bi)

        bv = jnp.full((LANES,), MINF, jnp.float32)
        bi = jnp.full((LANES,), BIG, jnp.int32)
        bv, bi = masked(first, bv, bi)

        def comb(a, b):
            av, ai = a
            bv_, bi_ = b
            upd = bv_ > av
            return jnp.maximum(av, bv_), jnp.where(upd, bi_, ai)

        g0 = first + 1
        n_int = jnp.maximum(last - g0, 0)
        ng = n_int >> 3

        def group(g, carry):
            goff = (g0 + g * 8) * LANES
            gp = base + goff
            leaves = [(vals_v[pl.ds(goff + k * LANES, LANES)],
                       gp + laneoffs[k]) for k in range(8)]
            l2 = [comb(leaves[k], leaves[k + 1]) for k in (0, 2, 4, 6)]
            l3 = [comb(l2[0], l2[1]), comb(l2[2], l2[3])]
            return comb(carry, comb(l3[0], l3[1]))

        bv, bi = lax.fori_loop(0, ng, group, (bv, bi))

        def interior(i, carry):
            cv, ci = carry
            off = i * LANES
            v = vals_v[pl.ds(off, LANES)]
            upd = v > cv
            return (jnp.where(upd, v, cv),
                    jnp.where(upd, base + off + lane, ci))

        bv, bi = lax.fori_loop(g0 + ng * 8, last, interior, (bv, bi))
        bv, bi = masked(last, bv, bi)

        mx = jnp.max(bv)
        mi = jnp.min(jnp.where(bv == mx, bi, BIG))
        sel = lane == svec
        return (jnp.where(sel, mx, my_vals),
                jnp.where(sel, mi, my_idxs))

    my_vals = jnp.full((LANES,), MINF, jnp.float32)
    my_idxs = jnp.full((LANES,), BIG, jnp.int32)
    my_vals, my_idxs = lax.fori_loop(
        seg_a, seg_b, per_segment, (my_vals, my_idxs))

    my_vals_v[...] = my_vals
    my_idxs_v[...] = my_idxs
    p1 = pltpu.async_copy(
        my_vals_v, shared_vals.at[pl.ds(tid * LANES, LANES)], sem)
    p2 = pltpu.async_copy(
        my_idxs_v, shared_idxs.at[pl.ds(tid * LANES, LANES)], sem)
    p1.wait()
    p2.wait()
    plsc.subcore_barrier()

    @pl.when(tid == 0)
    def _merge():
        m1 = pltpu.async_copy(shared_vals, merge_vals_v, sem)
        m2 = pltpu.async_copy(shared_idxs, merge_idxs_v, sem)
        m1.wait()
        m2.wait()
        acc_v = merge_vals_v[pl.ds(0, LANES)]
        acc_i = merge_idxs_v[pl.ds(0, LANES)]
        for r in range(1, NTILES):
            row_v = merge_vals_v[pl.ds(r * LANES, LANES)]
            row_i = merge_idxs_v[pl.ds(r * LANES, LANES)]
            upd = row_v > acc_v
            acc_v = jnp.where(upd, row_v, acc_v)
            acc_i = jnp.where(upd, row_i, acc_i)
        out_v[...] = jnp.where(ends_vec > starts_vec, acc_i, BIG)
        pltpu.sync_copy(out_v, out_hbm)


@functools.lru_cache(maxsize=1)
def _build():
  return pl.kernel(
    _body,
    out_type=jax.ShapeDtypeStruct((NSEG,), jnp.int32),
    mesh=plsc.VectorSubcoreMesh(
        core_axis_name="c", subcore_axis_name="s",
        num_cores=1, num_subcores=NTILES),
    scratch_types=[
        pltpu.VMEM((CHUNK,), jnp.float32),
        pltpu.VMEM((NSEG,), jnp.int32),
        pltpu.VMEM((NSEG,), jnp.int32),
        pltpu.VMEM((LANES,), jnp.float32),
        pltpu.VMEM((LANES,), jnp.int32),
        pltpu.VMEM_SHARED((NTILES * LANES,), jnp.float32),
        pltpu.VMEM_SHARED((NTILES * LANES,), jnp.int32),
        pltpu.VMEM((NTILES * LANES,), jnp.float32),
        pltpu.VMEM((NTILES * LANES,), jnp.int32),
        pltpu.VMEM((NSEG,), jnp.int32),
        pltpu.SemaphoreType.DMA,
    ],
    compiler_params=pltpu.CompilerParams(needs_layout_passes=False),
  )


def kernel(values, prefix_sum):
    ps = prefix_sum.astype(jnp.int32)
    starts = jnp.concatenate([jnp.zeros((1,), jnp.int32), ps[:-1]])
    out = _build()(values, starts, ps)
    return out.astype(jnp.int64)


# speedup vs baseline: 1.2754x; 1.0063x over previous
import functools

import jax
import jax.numpy as jnp
from jax import lax
from jax.experimental import pallas as pl
from jax.experimental.pallas import tpu as pltpu
from jax.experimental.pallas import tpu_sc as plsc

import numpy as np

TOTAL = 32768
NSEG = 16
NTILES = 16
CHUNK = TOTAL // NTILES
LANES = 16
VREGS = CHUNK // LANES

MINF = np.float32(float("-inf"))
BIG = np.int32(2147483647)


def _body(values_hbm, bounds_hbm, out_hbm,
          vals_v, bounds_v, pub_v, shared_cand, merge_v, out_v, sem):
    tid = lax.axis_index("s")
    base = tid * CHUNK

    c1 = pltpu.async_copy(values_hbm.at[pl.ds(base, CHUNK)], vals_v, sem)
    c2 = pltpu.async_copy(bounds_hbm, bounds_v, sem)
    c2.wait()
    c1.wait()

    lane = lax.iota(jnp.int32, LANES)
    starts_vec = bounds_v[pl.ds(0, LANES)]
    ends_vec = bounds_v[pl.ds(LANES, LANES)]
    laneoffs = [lane + np.int32(k * LANES) for k in range(8)]

    seg_a = jnp.sum((ends_vec <= base).astype(jnp.int32))
    seg_b = jnp.sum((starts_vec < base + CHUNK).astype(jnp.int32))

    def per_segment(s, carry):
        my_vals, my_idxs = carry
        svec = jnp.full((LANES,), s, jnp.int32)
        lo = plsc.load_gather(bounds_v, [svec])[0]
        hi = plsc.load_gather(bounds_v, [svec + LANES])[0]
        n0 = jnp.clip(lo - base, 0, CHUNK)
        n1 = jnp.clip(hi - base, 0, CHUNK)
        first = jnp.minimum(n0 >> 4, VREGS - 1)
        last = jnp.maximum((n1 - 1) >> 4, 0)

        def masked(i, bv, bi):
            off = i * LANES
            pos = base + off + lane
            v = vals_v[pl.ds(off, LANES)]
            vm = jnp.where((pos >= lo) & (pos < hi), v, MINF)
            upd = vm > bv
            return jnp.where(upd, vm, bv), jnp.where(upd, pos, bi)

        bv = jnp.full((LANES,), MINF, jnp.float32)
        bi = jnp.full((LANES,), BIG, jnp.int32)
        bv, bi = masked(first, bv, bi)

        def comb(a, b):
            av, ai = a
            bv_, bi_ = b
            upd = bv_ > av
            return jnp.maximum(av, bv_), jnp.where(upd, bi_, ai)

        g0 = first + 1
        n_int = jnp.maximum(last - g0, 0)
        ng = n_int >> 3

        def group(g, carry):
            goff = (g0 + g * 8) * LANES
            gp = base + goff
            leaves = [(vals_v[pl.ds(goff + k * LANES, LANES)],
                       gp + laneoffs[k]) for k in range(8)]
            l2 = [comb(leaves[k], leaves[k + 1]) for k in (0, 2, 4, 6)]
            l3 = [comb(l2[0], l2[1]), comb(l2[2], l2[3])]
            return comb(carry, comb(l3[0], l3[1]))

        bv, bi = lax.fori_loop(0, ng, group, (bv, bi))

        def interior(i, carry):
            cv, ci = carry
            off = i * LANES
            v = vals_v[pl.ds(off, LANES)]
            upd = v > cv
            return (jnp.where(upd, v, cv),
                    jnp.where(upd, base + off + lane, ci))

        bv, bi = lax.fori_loop(g0 + ng * 8, last, interior, (bv, bi))
        bv, bi = masked(last, bv, bi)

        mx = jnp.max(bv)
        mi = jnp.min(jnp.where(bv == mx, bi, BIG))
        sel = lane == svec
        return (jnp.where(sel, mx, my_vals),
                jnp.where(sel, mi, my_idxs))

    my_vals = jnp.full((LANES,), MINF, jnp.float32)
    my_idxs = jnp.full((LANES,), BIG, jnp.int32)
    my_vals, my_idxs = lax.fori_loop(
        seg_a, seg_b, per_segment, (my_vals, my_idxs))

    pub_v[pl.ds(0, LANES)] = plsc.bitcast(my_vals, jnp.int32)
    pub_v[pl.ds(LANES, LANES)] = my_idxs
    pltpu.sync_copy(pub_v, shared_cand.at[pl.ds(tid * 2 * LANES, 2 * LANES)])
    plsc.subcore_barrier()

    @pl.when(tid == 0)
    def _merge():
        pltpu.sync_copy(shared_cand, merge_v)
        acc_v = plsc.bitcast(merge_v[pl.ds(0, LANES)], jnp.float32)
        acc_i = merge_v[pl.ds(LANES, LANES)]
        for r in range(1, NTILES):
            row_v = plsc.bitcast(
                merge_v[pl.ds(r * 2 * LANES, LANES)], jnp.float32)
            row_i = merge_v[pl.ds(r * 2 * LANES + LANES, LANES)]
            upd = row_v > acc_v
            acc_v = jnp.where(upd, row_v, acc_v)
            acc_i = jnp.where(upd, row_i, acc_i)
        out_v[...] = jnp.where(ends_vec > starts_vec, acc_i, BIG)
        pltpu.sync_copy(out_v, out_hbm)


@functools.lru_cache(maxsize=1)
def _build():
  return pl.kernel(
    _body,
    out_type=jax.ShapeDtypeStruct((NSEG,), jnp.int32),
    mesh=plsc.VectorSubcoreMesh(
        core_axis_name="c", subcore_axis_name="s",
        num_cores=1, num_subcores=NTILES),
    scratch_types=[
        pltpu.VMEM((CHUNK,), jnp.float32),
        pltpu.VMEM((2 * NSEG,), jnp.int32),
        pltpu.VMEM((2 * LANES,), jnp.int32),
        pltpu.VMEM_SHARED((NTILES * 2 * LANES,), jnp.int32),
        pltpu.VMEM((NTILES * 2 * LANES,), jnp.int32),
        pltpu.VMEM((NSEG,), jnp.int32),
        pltpu.SemaphoreType.DMA,
    ],
    compiler_params=pltpu.CompilerParams(needs_layout_passes=False),
  )


def kernel(values, prefix_sum):
    ps = prefix_sum.astype(jnp.int32)
    starts = jnp.concatenate([jnp.zeros((1,), jnp.int32), ps[:-1]])
    bounds = jnp.concatenate([starts, ps])
    out = _build()(values, bounds)
    return out.astype(jnp.int64)


# speedup vs baseline: 1.2842x; 1.0069x over previous
import functools

import jax
import jax.numpy as jnp
from jax import lax
from jax.experimental import pallas as pl
from jax.experimental.pallas import tpu as pltpu
from jax.experimental.pallas import tpu_sc as plsc

import numpy as np

TOTAL = 32768
NSEG = 16
NTILES = 16
CHUNK = TOTAL // NTILES
LANES = 16
VREGS = CHUNK // LANES

MINF = np.float32(float("-inf"))
BIG = np.int32(2147483647)


def _body(values_hbm, ps_hbm, out_hbm,
          vals_v, ends_v, starts_sc_v, pub_v, shared_cand, merge_v, out_v,
          sem):
    tid = lax.axis_index("s")
    base = tid * CHUNK

    c1 = pltpu.async_copy(values_hbm.at[pl.ds(base, CHUNK)], vals_v, sem)
    c2 = pltpu.async_copy(ps_hbm, ends_v, sem)
    c2.wait()
    c1.wait()

    lane = lax.iota(jnp.int32, LANES)
    ends_vec = ends_v[...]
    shifted = plsc.load_gather(ends_v, [jnp.maximum(lane - 1, 0)])
    starts_vec = jnp.where(lane == 0, 0, shifted)
    starts_sc_v[...] = starts_vec
    laneoffs = [lane + np.int32(k * LANES) for k in range(8)]

    seg_a = jnp.sum((ends_vec <= base).astype(jnp.int32))
    seg_b = jnp.sum((starts_vec < base + CHUNK).astype(jnp.int32))

    def per_segment(s, carry):
        my_vals, my_idxs = carry
        svec = jnp.full((LANES,), s, jnp.int32)
        lo = plsc.load_gather(starts_sc_v, [svec])[0]
        hi = plsc.load_gather(ends_v, [svec])[0]
        n0 = jnp.clip(lo - base, 0, CHUNK)
        n1 = jnp.clip(hi - base, 0, CHUNK)
        first = jnp.minimum(n0 >> 4, VREGS - 1)
        last = jnp.maximum((n1 - 1) >> 4, 0)

        def masked(i, bv, bi):
            off = i * LANES
            pos = base + off + lane
            v = vals_v[pl.ds(off, LANES)]
            vm = jnp.where((pos >= lo) & (pos < hi), v, MINF)
            upd = vm > bv
            return jnp.where(upd, vm, bv), jnp.where(upd, pos, bi)

        bv = jnp.full((LANES,), MINF, jnp.float32)
        bi = jnp.full((LANES,), BIG, jnp.int32)
        bv, bi = masked(first, bv, bi)

        def comb(a, b):
            av, ai = a
            bv_, bi_ = b
            upd = bv_ > av
            return jnp.maximum(av, bv_), jnp.where(upd, bi_, ai)

        g0 = first + 1
        n_int = jnp.maximum(last - g0, 0)
        ng = n_int >> 3

        def group(g, carry):
            goff = (g0 + g * 8) * LANES
            gp = base + goff
            leaves = [(vals_v[pl.ds(goff + k * LANES, LANES)],
                       gp + laneoffs[k]) for k in range(8)]
            l2 = [comb(leaves[k], leaves[k + 1]) for k in (0, 2, 4, 6)]
            l3 = [comb(l2[0], l2[1]), comb(l2[2], l2[3])]
            return comb(carry, comb(l3[0], l3[1]))

        bv, bi = lax.fori_loop(0, ng, group, (bv, bi))

        def interior(i, carry):
            cv, ci = carry
            off = i * LANES
            v = vals_v[pl.ds(off, LANES)]
            upd = v > cv
            return (jnp.where(upd, v, cv),
                    jnp.where(upd, base + off + lane, ci))

        bv, bi = lax.fori_loop(g0 + ng * 8, last, interior, (bv, bi))
        bv, bi = masked(last, bv, bi)

        mx = jnp.max(bv)
        mi = jnp.min(jnp.where(bv == mx, bi, BIG))
        sel = lane == svec
        return (jnp.where(sel, mx, my_vals),
                jnp.where(sel, mi, my_idxs))

    my_vals = jnp.full((LANES,), MINF, jnp.float32)
    my_idxs = jnp.full((LANES,), BIG, jnp.int32)
    my_vals, my_idxs = lax.fori_loop(
        seg_a, seg_b, per_segment, (my_vals, my_idxs))

    pub_v[pl.ds(0, LANES)] = plsc.bitcast(my_vals, jnp.int32)
    pub_v[pl.ds(LANES, LANES)] = my_idxs
    pltpu.sync_copy(pub_v, shared_cand.at[pl.ds(tid * 2 * LANES, 2 * LANES)])
    plsc.subcore_barrier()

    @pl.when(tid == 0)
    def _merge():
        pltpu.sync_copy(shared_cand, merge_v)
        acc_v = plsc.bitcast(merge_v[pl.ds(0, LANES)], jnp.float32)
        acc_i = merge_v[pl.ds(LANES, LANES)]
        for r in range(1, NTILES):
            row_v = plsc.bitcast(
                merge_v[pl.ds(r * 2 * LANES, LANES)], jnp.float32)
            row_i = merge_v[pl.ds(r * 2 * LANES + LANES, LANES)]
            upd = row_v > acc_v
            acc_v = jnp.where(upd, row_v, acc_v)
            acc_i = jnp.where(upd, row_i, acc_i)
        out_v[...] = jnp.where(ends_vec > starts_vec, acc_i, BIG)
        pltpu.sync_copy(out_v, out_hbm)


@functools.lru_cache(maxsize=1)
def _build():
  return pl.kernel(
    _body,
    out_type=jax.ShapeDtypeStruct((NSEG,), jnp.int32),
    mesh=plsc.VectorSubcoreMesh(
        core_axis_name="c", subcore_axis_name="s",
        num_cores=1, num_subcores=NTILES),
    scratch_types=[
        pltpu.VMEM((CHUNK,), jnp.float32),
        pltpu.VMEM((NSEG,), jnp.int32),
        pltpu.VMEM((NSEG,), jnp.int32),
        pltpu.VMEM((2 * LANES,), jnp.int32),
        pltpu.VMEM_SHARED((NTILES * 2 * LANES,), jnp.int32),
        pltpu.VMEM((NTILES * 2 * LANES,), jnp.int32),
        pltpu.VMEM((NSEG,), jnp.int32),
        pltpu.SemaphoreType.DMA,
    ],
    compiler_params=pltpu.CompilerParams(needs_layout_passes=False),
  )


def kernel(values, prefix_sum):
    ps = prefix_sum.astype(jnp.int32)
    out = _build()(values, ps)
    return out.astype(jnp.int64)
